# Initial kernel scaffold; baseline (speedup 1.0000x reference)
import jax
import jax.numpy as jnp
from jax.experimental import pallas as pl


def kernel(z, params, pos_lat, pos_mid, pos_full, edge_lat, edge_mid, edge_full):
    raise NotImplementedError("write your pallas kernel here")


# speedup vs baseline: 2.5903x; 2.5903x over previous
import functools

import jax
import jax.numpy as jnp
from jax import lax
from jax.experimental import pallas as pl
from jax.experimental.pallas import tpu as pltpu
from jax.experimental.pallas import tpu_sc as plsc

F32 = jnp.float32
I32 = jnp.int32

_SELU_A = 1.6732632423543772
_SELU_S = 1.0507009873554805

_NC, _NS = 2, 16
_NW = _NC * _NS

_N_LAT, _N_MID, _N_FULL = 625, 2500, 10000
_NP_LAT, _NP_MID, _NP_FULL = 640, 2560, 10240


def _selu(x):
    return _SELU_S * jnp.where(x > 0, x, _SELU_A * (jnp.exp(x) - 1.0))


def _bm(N, target=512):
    return N if N <= 1280 else target


def _up_mlp(WupT, z_row, bup_col, Wl1, bl1, Wl2, bl2):
    def body(wt, zr, bc, w1, b1, w2, b2, o):
        h = jnp.sum(wt[...] * zr[...], axis=1, keepdims=True) + bc[...]
        a = _selu(h * w1[...] + b1[...])
        o[...] = jnp.dot(a, w2[...], preferred_element_type=F32) + b2[...]
    return pl.pallas_call(
        body,
        out_shape=jax.ShapeDtypeStruct((_NP_LAT, 128), F32),
    )(WupT, z_row, bup_col, Wl1, bl1, Wl2, bl2)


def _mm(x, Ws, bs, act=False, bm=512):
    N, K = x.shape
    m = len(Ws)
    bm = _bm(N, bm)

    def body(x_ref, *refs):
        outs = refs[2 * m:]
        xv = x_ref[...]
        for i in range(m):
            o = jnp.dot(xv, refs[i][...], preferred_element_type=F32) + refs[m + i][...]
            outs[i][...] = _selu(o) if act else o

    return pl.pallas_call(
        body,
        grid=(N // bm,),
        in_specs=[pl.BlockSpec((bm, K), lambda i: (i, 0))]
        + [pl.BlockSpec(W.shape, lambda i: (0, 0)) for W in Ws]
        + [pl.BlockSpec(b.shape, lambda i: (0, 0)) for b in bs],
        out_specs=[pl.BlockSpec((bm, W.shape[1]), lambda i: (i, 0)) for W in Ws],
        out_shape=[jax.ShapeDtypeStruct((N, W.shape[1]), F32) for W in Ws],
    )(x, *Ws, *bs)


def _combine_selu(S, parts, bm=512):
    N, F = S.shape
    bm = _bm(N, bm)

    def body(s_ref, p_ref, o_ref):
        o_ref[...] = _selu(s_ref[...] + p_ref[0] + p_ref[1])

    return pl.pallas_call(
        body,
        grid=(N // bm,),
        in_specs=[pl.BlockSpec((bm, F), lambda i: (i, 0)),
                  pl.BlockSpec((2, bm, F), lambda i: (0, i, 0))],
        out_specs=pl.BlockSpec((bm, F), lambda i: (i, 0)),
        out_shape=jax.ShapeDtypeStruct((N, F), F32),
    )(S, parts)


def _add_selu(a, b, bm=512):
    N, F = a.shape
    bm = _bm(N, bm)

    def body(a_ref, b_ref, o_ref):
        o_ref[...] = _selu(a_ref[...] + b_ref[...])

    return pl.pallas_call(
        body,
        grid=(N // bm,),
        in_specs=[pl.BlockSpec((bm, F), lambda i: (i, 0)),
                  pl.BlockSpec((bm, F), lambda i: (i, 0))],
        out_specs=pl.BlockSpec((bm, F), lambda i: (i, 0)),
        out_shape=jax.ShapeDtypeStruct((N, F), F32),
    )(a, b)


def _knn3(pn_cols, po_rows, Nn, bm=256):
    No = po_rows[0].shape[1]

    def body(px, py, pz, ox, oy, oz, i0, i1, i2, w0, w1, w2):
        d2 = ((px[...] - ox[...]) ** 2
              + (py[...] - oy[...]) ** 2
              + (pz[...] - oz[...]) ** 2)
        colid = lax.broadcasted_iota(I32, d2.shape, 1)
        idxs, vals = [], []
        for _ in range(3):
            m = jnp.min(d2, axis=1, keepdims=True)
            hit = d2 == m
            idx = jnp.min(jnp.where(hit, colid, jnp.int32(2**31 - 1)),
                          axis=1, keepdims=True)
            d2 = jnp.where(colid == idx, jnp.float32(3e38), d2)
            idxs.append(idx)
            vals.append(m)
        ws = [1.0 / (v + 1e-8) for v in vals]
        tot = ws[0] + ws[1] + ws[2]
        i0[...], i1[...], i2[...] = idxs
        w0[...] = ws[0] / tot
        w1[...] = ws[1] / tot
        w2[...] = ws[2] / tot

    col = pl.BlockSpec((bm, 1), lambda i: (i, 0))
    row = pl.BlockSpec((1, No), lambda i: (0, 0))
    return pl.pallas_call(
        body,
        grid=(Nn // bm,),
        in_specs=[col] * 3 + [row] * 3,
        out_specs=[col] * 6,
        out_shape=[jax.ShapeDtypeStruct((Nn, 1), I32)] * 3
        + [jax.ShapeDtypeStruct((Nn, 1), F32)] * 3,
    )(*pn_cols, *po_rows)


def _wsum(Gs, ws, bm=512):
    m = len(Gs)
    _, N, F = Gs[0].shape
    bm = _bm(N, bm)

    def body(*refs):
        g_refs = refs[:m]
        w_refs = refs[m:m + 3]
        outs = refs[m + 3:]
        for j in range(m):
            g = g_refs[j]
            outs[j][...] = (w_refs[0][...] * g[0] + w_refs[1][...] * g[1]
                            + w_refs[2][...] * g[2])

    return pl.pallas_call(
        body,
        grid=(N // bm,),
        in_specs=[pl.BlockSpec((3, bm, F), lambda i: (0, i, 0))] * m
        + [pl.BlockSpec((bm, 1), lambda i: (i, 0))] * 3,
        out_specs=[pl.BlockSpec((bm, F), lambda i: (i, 0))] * m,
        out_shape=[jax.ShapeDtypeStruct((N, F), F32)] * m,
    )(*Gs, *ws)


def _wsum_mm(G, ws, Wn, Wsk, b, bm=512):
    _, N, F = G.shape
    Fout = Wn.shape[1]
    bm = _bm(N, bm)

    def body(g_ref, w0, w1, w2, wn, wsk, b_ref, to, so):
        xin = (w0[...] * g_ref[0] + w1[...] * g_ref[1] + w2[...] * g_ref[2])
        to[...] = jnp.dot(xin, wn[...], preferred_element_type=F32)
        so[...] = jnp.dot(xin, wsk[...], preferred_element_type=F32) + b_ref[...]

    col = pl.BlockSpec((bm, 1), lambda i: (i, 0))
    return pl.pallas_call(
        body,
        grid=(N // bm,),
        in_specs=[pl.BlockSpec((3, bm, F), lambda i: (0, i, 0)), col, col, col,
                  pl.BlockSpec(Wn.shape, lambda i: (0, 0)),
                  pl.BlockSpec(Wsk.shape, lambda i: (0, 0)),
                  pl.BlockSpec(b.shape, lambda i: (0, 0))],
        out_specs=[pl.BlockSpec((bm, Fout), lambda i: (i, 0))] * 2,
        out_shape=[jax.ShapeDtypeStruct((N, Fout), F32)] * 2,
    )(G, *ws, Wn, Wsk, b)


def _decoder(x, oW1, ob1, oW2p, ob2p, gp, betap, bm=512):
    N, _ = x.shape
    bm = _bm(N, bm)

    def body(x_ref, w1, b1, w2, b2, g_ref, be_ref, o_ref):
        h = _selu(jnp.dot(x_ref[...], w1[...], preferred_element_type=F32) + b1[...])
        o = jnp.dot(h, w2[...], preferred_element_type=F32) + b2[...]
        lane = lax.broadcasted_iota(I32, o.shape, 1)
        msk = (lane < 3).astype(F32)
        mu = jnp.sum(o * msk, axis=1, keepdims=True) / 3.0
        oc = (o - mu) * msk
        var = jnp.sum(oc * oc, axis=1, keepdims=True) / 3.0
        o_ref[...] = oc / jnp.sqrt(var + 1e-5) * g_ref[...] + be_ref[...]

    return pl.pallas_call(
        body,
        grid=(N // bm,),
        in_specs=[pl.BlockSpec((bm, 128), lambda i: (i, 0)),
                  pl.BlockSpec(oW1.shape, lambda i: (0, 0)),
                  pl.BlockSpec(ob1.shape, lambda i: (0, 0)),
                  pl.BlockSpec(oW2p.shape, lambda i: (0, 0)),
                  pl.BlockSpec(ob2p.shape, lambda i: (0, 0)),
                  pl.BlockSpec(gp.shape, lambda i: (0, 0)),
                  pl.BlockSpec(betap.shape, lambda i: (0, 0))],
        out_specs=pl.BlockSpec((bm, 128), lambda i: (i, 0)),
        out_shape=jax.ShapeDtypeStruct((N, 128), F32),
    )(x, oW1, ob1, oW2p, ob2p, gp, betap)


def _chunk(bpw):
    for ch in range(128, 0, -8):
        if bpw % ch == 0:
            return ch
    raise ValueError(bpw)


def _sc_gather(table, idx):
    NT, F = table.shape
    B = idx.shape[0]
    bpw = B // _NW
    ch = _chunk(bpw)
    n = bpw // ch
    mesh = plsc.VectorSubcoreMesh(core_axis_name="c", subcore_axis_name="s")

    @functools.partial(
        pl.kernel, mesh=mesh,
        out_type=jax.ShapeDtypeStruct((B, F), F32),
        scratch_types=[pltpu.VMEM((bpw,), I32),
                       pltpu.VMEM((ch, F), F32),
                       pltpu.SemaphoreType.DMA],
    )
    def k(table_hbm, idx_hbm, out_hbm, idx_v, rows_v, sem):
        wid = lax.axis_index("s") * _NC + lax.axis_index("c")
        base = wid * bpw
        pltpu.sync_copy(idx_hbm.at[pl.ds(base, bpw)], idx_v)
        for c in range(n):
            pltpu.async_copy(
                table_hbm.at[idx_v.at[pl.ds(c * ch, ch)]], rows_v, sem).wait()
            pltpu.sync_copy(rows_v, out_hbm.at[pl.ds(base + c * ch, ch)])

    return k(table, idx)


def _sc_segsum(table, src, dst):
    NT, F = table.shape
    nb = F // 128
    M = NT * nb
    tflat = table.reshape(M, 128)
    srcs = jnp.stack([src * nb + cb for cb in range(nb)])
    dsts = jnp.stack([dst * nb + cb for cb in range(nb)])
    zeros = jnp.zeros((M, 128), F32)
    E = src.shape[0]
    epw = E // _NW
    ch = 128
    n = epw // ch
    ztr = M // _NS
    mesh = plsc.VectorSubcoreMesh(core_axis_name="c", subcore_axis_name="s")

    @functools.partial(
        pl.kernel, mesh=mesh,
        out_type=jax.ShapeDtypeStruct((_NC, M, 128), F32),
        scratch_types=[pltpu.VMEM((ch,), I32),
                       pltpu.VMEM((ch,), I32),
                       pltpu.VMEM((ch, 128), F32),
                       pltpu.VMEM_SHARED((M, 128), F32),
                       pltpu.SemaphoreType.DMA],
    )
    def k(table_hbm, src_hbm, dst_hbm, zero_hbm, out_hbm,
          srcv, dstv, rows, acc, sem):
        cid = lax.axis_index("c")
        sid = lax.axis_index("s")
        base = (sid * _NC + cid) * epw
        pltpu.sync_copy(zero_hbm.at[pl.ds(sid * ztr, ztr)],
                        acc.at[pl.ds(sid * ztr, ztr)])
        plsc.subcore_barrier()

        def body(c, carry):
            off = base + c * ch
            for cb in range(nb):
                pltpu.sync_copy(src_hbm.at[cb, pl.ds(off, ch)], srcv)
                pltpu.sync_copy(dst_hbm.at[cb, pl.ds(off, ch)], dstv)
                pltpu.async_copy(table_hbm.at[srcv], rows, sem).wait()
                pltpu.sync_copy(rows, acc.at[dstv], add=True)
            return carry

        lax.fori_loop(0, n, body, 0)
        plsc.subcore_barrier()
        pltpu.sync_copy(acc.at[pl.ds(sid * ztr, ztr)],
                        out_hbm.at[cid, pl.ds(sid * ztr, ztr)])

    return k(tflat, srcs, dsts, zeros).reshape(_NC, NT, F)


def _pad_rows(a, n):
    return jnp.pad(a, ((0, n - a.shape[0]), (0, 0)))


def _pad_edges(e, ep, dump):
    E = e.shape[1]
    src = jnp.concatenate([e[0], jnp.zeros((ep - E,), I32)])
    dst = jnp.concatenate([e[1], jnp.full((ep - E,), dump, I32)])
    return src, dst


def _pos_split(pos, n, padval, rows=False):
    p = jnp.pad(pos, ((0, n - pos.shape[0]), (0, 0)),
                constant_values=padval)
    cols = [p[:, d:d + 1] for d in range(3)]
    return [c.T for c in cols] if rows else cols


def _idx_cat(idxs):
    return jnp.concatenate([i[:, 0] for i in idxs])


def _mpl_here(x, Wn, Ws, b, src, dst):
    T, S = _mm(x, [Wn, Ws], [jnp.zeros((1, Wn.shape[1]), F32), b])
    parts = _sc_segsum(T, src, dst)
    return _combine_selu(S, parts)


def kernel(z, params, pos_lat, pos_mid, pos_full, edge_lat, edge_mid, edge_full):
    p = params
    b_ = {k: p[k].reshape(1, -1) for k in
          ("bl1", "bl2", "bb", "s0b", "m0ab", "m0bb", "s1b", "m1ab",
           "m1bb", "fb", "ob1")}

    src_l, dst_l = _pad_edges(edge_lat, 20480, _N_LAT)
    src_m, dst_m = _pad_edges(edge_mid, 81920, _N_MID)
    src_f, dst_f = _pad_edges(edge_full, 323584, _N_FULL)
    WupT = _pad_rows(p["Wup"].T, _NP_LAT)
    bup_col = _pad_rows(p["bup"].reshape(-1, 1), _NP_LAT)
    x = _up_mlp(WupT, z.reshape(1, 128), bup_col, p["Wl1"], b_["bl1"],
                p["Wl2"], b_["bl2"])

    x = _mpl_here(x, p["bWn"], p["bWs"], b_["bb"], src_l, dst_l)

    lat_old = _pos_split(pos_lat, _NP_LAT, 1e9, rows=True)
    mid_new = _pos_split(pos_mid, _NP_MID, 0.0)
    mid_old = _pos_split(pos_mid, _NP_MID, 1e9, rows=True)
    full_new = _pos_split(pos_full, _NP_FULL, 0.0)
    k0 = _knn3(mid_new, lat_old, _NP_MID)
    idx0_cat, w0 = _idx_cat(k0[:3]), list(k0[3:])
    k1 = _knn3(full_new, mid_old, _NP_FULL)
    idx1_cat, w1 = _idx_cat(k1[:3]), list(k1[3:])

    A, B = _mm(x, [p["s0Wn"], p["s0Ws"]],
               [jnp.zeros((1, 256), F32), b_["s0b"]])
    GA = _sc_gather(A, idx0_cat).reshape(3, _NP_MID, 256)
    GB = _sc_gather(B, idx0_cat).reshape(3, _NP_MID, 256)
    Tsk, Ssk = _wsum([GA, GB], w0)
    parts = _sc_segsum(Tsk, src_m, dst_m)
    skip = _combine_selu(Ssk, parts)

    y = _mpl_here(x, p["m0aWn"], p["m0aWs"], b_["m0ab"], src_l, dst_l)
    Gy = _sc_gather(y, idx0_cat).reshape(3, _NP_MID, 128)
    T, S = _wsum_mm(Gy, w0, p["m0bWn"], p["m0bWs"], b_["m0bb"])
    parts = _sc_segsum(T, src_m, dst_m)
    y = _combine_selu(S, parts)
    x = _add_selu(y, skip)

    A, B = _mm(x, [p["s1Wn"], p["s1Ws"]],
               [jnp.zeros((1, 128), F32), b_["s1b"]])
    GA = _sc_gather(A, idx1_cat).reshape(3, _NP_FULL, 128)
    GB = _sc_gather(B, idx1_cat).reshape(3, _NP_FULL, 128)
    Tsk, Ssk = _wsum([GA, GB], w1)
    parts = _sc_segsum(Tsk, src_f, dst_f)
    skip = _combine_selu(Ssk, parts)

    y = _mpl_here(x, p["m1aWn"], p["m1aWs"], b_["m1ab"], src_m, dst_m)
    Gy = _sc_gather(y, idx1_cat).reshape(3, _NP_FULL, 128)
    T, S = _wsum_mm(Gy, w1, p["m1bWn"], p["m1bWs"], b_["m1bb"])
    parts = _sc_segsum(T, src_f, dst_f)
    y = _combine_selu(S, parts)
    x = _add_selu(y, skip)

    x = _mpl_here(x, p["fWn"], p["fWs"], b_["fb"], src_f, dst_f)
    oW2p = jnp.pad(p["oW2"], ((0, 0), (0, 125)))
    ob2p = jnp.pad(p["ob2"].reshape(1, 3), ((0, 0), (0, 125)))
    gp = jnp.pad(p["g"].reshape(1, 3), ((0, 0), (0, 125)))
    betap = jnp.pad(p["beta"].reshape(1, 3), ((0, 0), (0, 125)))
    out = _decoder(x, p["oW1"], b_["ob1"], oW2p, ob2p, gp, betap)
    return out[:_N_FULL, :3]


# speedup vs baseline: 3.2055x; 1.2375x over previous
import functools

import jax
import jax.numpy as jnp
from jax import lax
from jax.experimental import pallas as pl
from jax.experimental.pallas import tpu as pltpu
from jax.experimental.pallas import tpu_sc as plsc

F32 = jnp.float32
I32 = jnp.int32

_SELU_A = 1.6732632423543772
_SELU_S = 1.0507009873554805

_NC, _NS = 2, 16
_NW = _NC * _NS

_N_LAT, _N_MID, _N_FULL = 625, 2500, 10000
_NP_LAT, _NP_MID, _NP_FULL = 640, 2560, 10240


def _selu(x):
    return _SELU_S * jnp.where(x > 0, x, _SELU_A * (jnp.exp(x) - 1.0))


def _bm(N, target=512):
    return N if N <= 1280 else target


def _up_mlp(WupT, z_row, bup_col, Wl1, bl1, Wl2, bl2):
    def body(wt, zr, bc, w1, b1, w2, b2, o):
        h = jnp.sum(wt[...] * zr[...], axis=1, keepdims=True) + bc[...]
        a = _selu(h * w1[...] + b1[...])
        o[...] = jnp.dot(a, w2[...], preferred_element_type=F32) + b2[...]
    return pl.pallas_call(
        body,
        out_shape=jax.ShapeDtypeStruct((_NP_LAT, 128), F32),
    )(WupT, z_row, bup_col, Wl1, bl1, Wl2, bl2)


def _mm(x, Ws, bs, act=False, bm=512):
    N, K = x.shape
    m = len(Ws)
    bm = _bm(N, bm)

    def body(x_ref, *refs):
        outs = refs[2 * m:]
        xv = x_ref[...]
        for i in range(m):
            o = jnp.dot(xv, refs[i][...], preferred_element_type=F32) + refs[m + i][...]
            outs[i][...] = _selu(o) if act else o

    return pl.pallas_call(
        body,
        grid=(N // bm,),
        in_specs=[pl.BlockSpec((bm, K), lambda i: (i, 0))]
        + [pl.BlockSpec(W.shape, lambda i: (0, 0)) for W in Ws]
        + [pl.BlockSpec(b.shape, lambda i: (0, 0)) for b in bs],
        out_specs=[pl.BlockSpec((bm, W.shape[1]), lambda i: (i, 0)) for W in Ws],
        out_shape=[jax.ShapeDtypeStruct((N, W.shape[1]), F32) for W in Ws],
    )(x, *Ws, *bs)


def _combine_selu(S, parts, bm=512):
    N, F = S.shape
    bm = _bm(N, bm)

    def body(s_ref, p_ref, o_ref):
        o_ref[...] = _selu(s_ref[...] + p_ref[0] + p_ref[1])

    return pl.pallas_call(
        body,
        grid=(N // bm,),
        in_specs=[pl.BlockSpec((bm, F), lambda i: (i, 0)),
                  pl.BlockSpec((2, bm, F), lambda i: (0, i, 0))],
        out_specs=pl.BlockSpec((bm, F), lambda i: (i, 0)),
        out_shape=jax.ShapeDtypeStruct((N, F), F32),
    )(S, parts)


def _add_selu(a, b, bm=512):
    N, F = a.shape
    bm = _bm(N, bm)

    def body(a_ref, b_ref, o_ref):
        o_ref[...] = _selu(a_ref[...] + b_ref[...])

    return pl.pallas_call(
        body,
        grid=(N // bm,),
        in_specs=[pl.BlockSpec((bm, F), lambda i: (i, 0)),
                  pl.BlockSpec((bm, F), lambda i: (i, 0))],
        out_specs=pl.BlockSpec((bm, F), lambda i: (i, 0)),
        out_shape=jax.ShapeDtypeStruct((N, F), F32),
    )(a, b)


def _knn3(pn_cols, po_rows, Nn, bm=256):
    No = po_rows[0].shape[1]

    def body(px, py, pz, ox, oy, oz, i0, i1, i2, w0, w1, w2):
        d2 = ((px[...] - ox[...]) ** 2
              + (py[...] - oy[...]) ** 2
              + (pz[...] - oz[...]) ** 2)
        colid = lax.broadcasted_iota(I32, d2.shape, 1)
        idxs, vals = [], []
        for _ in range(3):
            m = jnp.min(d2, axis=1, keepdims=True)
            hit = d2 == m
            idx = jnp.min(jnp.where(hit, colid, jnp.int32(2**31 - 1)),
                          axis=1, keepdims=True)
            d2 = jnp.where(colid == idx, jnp.float32(3e38), d2)
            idxs.append(idx)
            vals.append(m)
        ws = [1.0 / (v + 1e-8) for v in vals]
        tot = ws[0] + ws[1] + ws[2]
        i0[...], i1[...], i2[...] = idxs
        w0[...] = ws[0] / tot
        w1[...] = ws[1] / tot
        w2[...] = ws[2] / tot

    col = pl.BlockSpec((bm, 1), lambda i: (i, 0))
    row = pl.BlockSpec((1, No), lambda i: (0, 0))
    return pl.pallas_call(
        body,
        grid=(Nn // bm,),
        in_specs=[col] * 3 + [row] * 3,
        out_specs=[col] * 6,
        out_shape=[jax.ShapeDtypeStruct((Nn, 1), I32)] * 3
        + [jax.ShapeDtypeStruct((Nn, 1), F32)] * 3,
    )(*pn_cols, *po_rows)


def _wsum(Gs, ws, bm=512):
    m = len(Gs)
    _, N, F = Gs[0].shape
    bm = _bm(N, bm)

    def body(*refs):
        g_refs = refs[:m]
        w_refs = refs[m:m + 3]
        outs = refs[m + 3:]
        for j in range(m):
            g = g_refs[j]
            outs[j][...] = (w_refs[0][...] * g[0] + w_refs[1][...] * g[1]
                            + w_refs[2][...] * g[2])

    return pl.pallas_call(
        body,
        grid=(N // bm,),
        in_specs=[pl.BlockSpec((3, bm, F), lambda i: (0, i, 0))] * m
        + [pl.BlockSpec((bm, 1), lambda i: (i, 0))] * 3,
        out_specs=[pl.BlockSpec((bm, F), lambda i: (i, 0))] * m,
        out_shape=[jax.ShapeDtypeStruct((N, F), F32)] * m,
    )(*Gs, *ws)


def _wsum_mm(G, ws, Wn, Wsk, b, bm=512):
    _, N, F = G.shape
    Fout = Wn.shape[1]
    bm = _bm(N, bm)

    def body(g_ref, w0, w1, w2, wn, wsk, b_ref, to, so):
        xin = (w0[...] * g_ref[0] + w1[...] * g_ref[1] + w2[...] * g_ref[2])
        to[...] = jnp.dot(xin, wn[...], preferred_element_type=F32)
        so[...] = jnp.dot(xin, wsk[...], preferred_element_type=F32) + b_ref[...]

    col = pl.BlockSpec((bm, 1), lambda i: (i, 0))
    return pl.pallas_call(
        body,
        grid=(N // bm,),
        in_specs=[pl.BlockSpec((3, bm, F), lambda i: (0, i, 0)), col, col, col,
                  pl.BlockSpec(Wn.shape, lambda i: (0, 0)),
                  pl.BlockSpec(Wsk.shape, lambda i: (0, 0)),
                  pl.BlockSpec(b.shape, lambda i: (0, 0))],
        out_specs=[pl.BlockSpec((bm, Fout), lambda i: (i, 0))] * 2,
        out_shape=[jax.ShapeDtypeStruct((N, Fout), F32)] * 2,
    )(G, *ws, Wn, Wsk, b)


def _decoder(x, oW1, ob1, oW2p, ob2p, gp, betap, bm=512):
    N, _ = x.shape
    bm = _bm(N, bm)

    def body(x_ref, w1, b1, w2, b2, g_ref, be_ref, o_ref):
        h = _selu(jnp.dot(x_ref[...], w1[...], preferred_element_type=F32) + b1[...])
        o = jnp.dot(h, w2[...], preferred_element_type=F32) + b2[...]
        lane = lax.broadcasted_iota(I32, o.shape, 1)
        msk = (lane < 3).astype(F32)
        mu = jnp.sum(o * msk, axis=1, keepdims=True) / 3.0
        oc = (o - mu) * msk
        var = jnp.sum(oc * oc, axis=1, keepdims=True) / 3.0
        o_ref[...] = oc / jnp.sqrt(var + 1e-5) * g_ref[...] + be_ref[...]

    return pl.pallas_call(
        body,
        grid=(N // bm,),
        in_specs=[pl.BlockSpec((bm, 128), lambda i: (i, 0)),
                  pl.BlockSpec(oW1.shape, lambda i: (0, 0)),
                  pl.BlockSpec(ob1.shape, lambda i: (0, 0)),
                  pl.BlockSpec(oW2p.shape, lambda i: (0, 0)),
                  pl.BlockSpec(ob2p.shape, lambda i: (0, 0)),
                  pl.BlockSpec(gp.shape, lambda i: (0, 0)),
                  pl.BlockSpec(betap.shape, lambda i: (0, 0))],
        out_specs=pl.BlockSpec((bm, 128), lambda i: (i, 0)),
        out_shape=jax.ShapeDtypeStruct((N, 128), F32),
    )(x, oW1, ob1, oW2p, ob2p, gp, betap)


def _chunk(bpw):
    for ch in range(128, 0, -8):
        if bpw % ch == 0:
            return ch
    raise ValueError(bpw)


def _sc_gather(table, idx):
    NT, F = table.shape
    B = idx.shape[0]
    bpw = B // _NW
    ch = _chunk(bpw)
    n = bpw // ch
    mesh = plsc.VectorSubcoreMesh(core_axis_name="c", subcore_axis_name="s")

    @functools.partial(
        pl.kernel, mesh=mesh,
        out_type=jax.ShapeDtypeStruct((B, F), F32),
        scratch_types=[pltpu.VMEM((bpw,), I32),
                       pltpu.VMEM((ch, F), F32),
                       pltpu.SemaphoreType.DMA],
    )
    def k(table_hbm, idx_hbm, out_hbm, idx_v, rows_v, sem):
        wid = lax.axis_index("s") * _NC + lax.axis_index("c")
        base = wid * bpw
        pltpu.sync_copy(idx_hbm.at[pl.ds(base, bpw)], idx_v)
        for c in range(n):
            pltpu.async_copy(
                table_hbm.at[idx_v.at[pl.ds(c * ch, ch)]], rows_v, sem).wait()
            pltpu.sync_copy(rows_v, out_hbm.at[pl.ds(base + c * ch, ch)])

    return k(table, idx)


def _sc_segsum(table, src, dst):
    NT, F = table.shape
    nb = F // 128
    M = NT * nb
    tflat = table.reshape(M, 128)
    E = src.shape[0]
    ch = 128
    epw = E // _NW
    n = epw // ch
    J = n * nb
    srcs = jnp.stack([src * nb + cb for cb in range(nb)])
    dsts = jnp.stack([dst * nb + cb for cb in range(nb)])
    zeros = jnp.zeros((M, 128), F32)
    ztr = M // _NS
    mesh = plsc.VectorSubcoreMesh(core_axis_name="c", subcore_axis_name="s")

    @functools.partial(
        pl.kernel, mesh=mesh,
        out_type=jax.ShapeDtypeStruct((_NC, M, 128), F32),
        scratch_types=[pltpu.VMEM((ch,), I32), pltpu.VMEM((ch,), I32),
                       pltpu.VMEM((ch,), I32), pltpu.VMEM((ch,), I32),
                       pltpu.VMEM((ch, 128), F32),
                       pltpu.VMEM((ch, 128), F32),
                       pltpu.VMEM_SHARED((M, 128), F32),
                       pltpu.SemaphoreType.DMA,
                       pltpu.SemaphoreType.DMA],
    )
    def k(table_hbm, src_hbm, dst_hbm, zero_hbm, out_hbm,
          srcv0, srcv1, dstv0, dstv1, rows0, rows1, acc, sem0, sem1):
        cid = lax.axis_index("c")
        sid = lax.axis_index("s")
        base = (sid * _NC + cid) * epw
        srcv = (srcv0, srcv1)
        dstv = (dstv0, dstv1)
        rows = (rows0, rows1)
        sems = (sem0, sem1)
        pltpu.sync_copy(zero_hbm.at[pl.ds(sid * ztr, ztr)],
                        acc.at[pl.ds(sid * ztr, ztr)])
        plsc.subcore_barrier()

        def jcb(j):
            cb = j // n if isinstance(j, int) else lax.div(j, n)
            c = j - cb * n
            return cb, base + c * ch

        def start(j, b):
            cb, off = jcb(j)
            pltpu.sync_copy(src_hbm.at[cb, pl.ds(off, ch)], srcv[b])
            pltpu.sync_copy(dst_hbm.at[cb, pl.ds(off, ch)], dstv[b])
            pltpu.async_copy(table_hbm.at[srcv[b]], rows[b], sems[b])

        def drain(j, b):
            pltpu.make_async_copy(table_hbm.at[srcv[b]],
                                  rows[b], sems[b]).wait()
            pltpu.sync_copy(rows[b], acc.at[dstv[b]], add=True)

        start(0, 0)
        start(1, 1)
        I = (J - 2) // 2

        def body(i, carry):
            j0 = 2 * i
            drain(j0, 0)
            start(j0 + 2, 0)
            drain(j0 + 1, 1)
            start(j0 + 3, 1)
            return carry

        lax.fori_loop(0, I, body, 0)
        drain(2 * I, 0)
        drain(2 * I + 1, 1)
        if J % 2:
            start(2 * I + 2, 0)
            drain(2 * I + 2, 0)
        plsc.subcore_barrier()
        pltpu.sync_copy(acc.at[pl.ds(sid * ztr, ztr)],
                        out_hbm.at[cid, pl.ds(sid * ztr, ztr)])

    return k(tflat, srcs, dsts, zeros).reshape(_NC, NT, F)


def _pad_rows(a, n):
    return jnp.pad(a, ((0, n - a.shape[0]), (0, 0)))


def _pad_edges(e, ep, dump):
    E = e.shape[1]
    src = jnp.concatenate([e[0], jnp.zeros((ep - E,), I32)])
    dst = jnp.concatenate([e[1], jnp.full((ep - E,), dump, I32)])
    return src, dst


def _pos_split(pos, n, padval, rows=False):
    p = jnp.pad(pos, ((0, n - pos.shape[0]), (0, 0)),
                constant_values=padval)
    cols = [p[:, d:d + 1] for d in range(3)]
    return [c.T for c in cols] if rows else cols


def _idx_cat(idxs):
    return jnp.concatenate([i[:, 0] for i in idxs])


def _mpl_here(x, Wn, Ws, b, src, dst):
    T, S = _mm(x, [Wn, Ws], [jnp.zeros((1, Wn.shape[1]), F32), b])
    parts = _sc_segsum(T, src, dst)
    return _combine_selu(S, parts)


def kernel(z, params, pos_lat, pos_mid, pos_full, edge_lat, edge_mid, edge_full):
    p = params
    b_ = {k: p[k].reshape(1, -1) for k in
          ("bl1", "bl2", "bb", "s0b", "m0ab", "m0bb", "s1b", "m1ab",
           "m1bb", "fb", "ob1")}

    src_l, dst_l = _pad_edges(edge_lat, 20480, _N_LAT)
    src_m, dst_m = _pad_edges(edge_mid, 81920, _N_MID)
    src_f, dst_f = _pad_edges(edge_full, 323584, _N_FULL)
    WupT = _pad_rows(p["Wup"].T, _NP_LAT)
    bup_col = _pad_rows(p["bup"].reshape(-1, 1), _NP_LAT)
    x = _up_mlp(WupT, z.reshape(1, 128), bup_col, p["Wl1"], b_["bl1"],
                p["Wl2"], b_["bl2"])

    x = _mpl_here(x, p["bWn"], p["bWs"], b_["bb"], src_l, dst_l)

    lat_old = _pos_split(pos_lat, _NP_LAT, 1e9, rows=True)
    mid_new = _pos_split(pos_mid, _NP_MID, 0.0)
    mid_old = _pos_split(pos_mid, _NP_MID, 1e9, rows=True)
    full_new = _pos_split(pos_full, _NP_FULL, 0.0)
    k0 = _knn3(mid_new, lat_old, _NP_MID)
    idx0_cat, w0 = _idx_cat(k0[:3]), list(k0[3:])
    k1 = _knn3(full_new, mid_old, _NP_FULL)
    idx1_cat, w1 = _idx_cat(k1[:3]), list(k1[3:])

    A, B = _mm(x, [p["s0Wn"], p["s0Ws"]],
               [jnp.zeros((1, 256), F32), b_["s0b"]])
    GA = _sc_gather(A, idx0_cat).reshape(3, _NP_MID, 256)
    GB = _sc_gather(B, idx0_cat).reshape(3, _NP_MID, 256)
    Tsk, Ssk = _wsum([GA, GB], w0)
    parts = _sc_segsum(Tsk, src_m, dst_m)
    skip = _combine_selu(Ssk, parts)

    y = _mpl_here(x, p["m0aWn"], p["m0aWs"], b_["m0ab"], src_l, dst_l)
    Gy = _sc_gather(y, idx0_cat).reshape(3, _NP_MID, 128)
    T, S = _wsum_mm(Gy, w0, p["m0bWn"], p["m0bWs"], b_["m0bb"])
    parts = _sc_segsum(T, src_m, dst_m)
    y = _combine_selu(S, parts)
    x = _add_selu(y, skip)

    A, B = _mm(x, [p["s1Wn"], p["s1Ws"]],
               [jnp.zeros((1, 128), F32), b_["s1b"]])
    GA = _sc_gather(A, idx1_cat).reshape(3, _NP_FULL, 128)
    GB = _sc_gather(B, idx1_cat).reshape(3, _NP_FULL, 128)
    Tsk, Ssk = _wsum([GA, GB], w1)
    parts = _sc_segsum(Tsk, src_f, dst_f)
    skip = _combine_selu(Ssk, parts)

    y = _mpl_here(x, p["m1aWn"], p["m1aWs"], b_["m1ab"], src_m, dst_m)
    Gy = _sc_gather(y, idx1_cat).reshape(3, _NP_FULL, 128)
    T, S = _wsum_mm(Gy, w1, p["m1bWn"], p["m1bWs"], b_["m1bb"])
    parts = _sc_segsum(T, src_f, dst_f)
    y = _combine_selu(S, parts)
    x = _add_selu(y, skip)

    x = _mpl_here(x, p["fWn"], p["fWs"], b_["fb"], src_f, dst_f)
    oW2p = jnp.pad(p["oW2"], ((0, 0), (0, 125)))
    ob2p = jnp.pad(p["ob2"].reshape(1, 3), ((0, 0), (0, 125)))
    gp = jnp.pad(p["g"].reshape(1, 3), ((0, 0), (0, 125)))
    betap = jnp.pad(p["beta"].reshape(1, 3), ((0, 0), (0, 125)))
    out = _decoder(x, p["oW1"], b_["ob1"], oW2p, ob2p, gp, betap)
    return out[:_N_FULL, :3]
